# baseline (speedup 1.0000x reference)
import functools

import jax
import jax.numpy as jnp
from jax import lax
from jax.experimental import pallas as pl
from jax.experimental.pallas import tpu as pltpu
from jax.experimental.pallas import tpu_sc as plsc

B = 1024
CONTEXT_LEN = 77
CTX_DIM = 512
N_CTX = 8
NC, NS = 2, 16
NW = NC * NS
LPW = B // NW
SUF = CONTEXT_LEN - N_CTX - 1
GW = SUF + 4
TCB = 8


def _sc_body(labels_hbm, perm_hbm, table_hbm, tp1d_hbm, g1_hbm,
             labels_v, perm_v, tok0, tok1, s0, s1,
             gsem0, gsem1, wsem0, wsem1):
    toks = (tok0, tok1)
    stage = (s0, s1)
    gsem = (gsem0, gsem1)
    wsem = (wsem0, wsem1)
    wid = lax.axis_index("s") * NC + lax.axis_index("c")
    base = wid * LPW
    pltpu.sync_copy(labels_hbm.at[pl.ds(base, LPW)], labels_v)
    pltpu.sync_copy(perm_hbm.at[pl.ds(base, LPW)], perm_v)
    lanes = lax.iota(jnp.int32, 16)

    def extract(vec, i):
        chunk = vec[pl.ds(16 * (i // 16), 16)]
        return jnp.max(jnp.where(lanes == (i % 16), chunk, 0))

    def issue_gather(i):
        p = i % 2
        lbl = extract(labels_v, i)
        pltpu.sync_copy(tp1d_hbm.at[pl.ds(lbl * GW, GW)], toks[p])
        pltpu.async_copy(table_hbm.at[toks[p]], stage[p], gsem[p])

    def wait_gather(p):
        pltpu.make_async_copy(table_hbm.at[pl.ds(0, GW)], stage[p],
                              gsem[p]).wait()

    def issue_write(i):
        p = i % 2
        pltpu.async_copy(stage[p], g1_hbm.at[extract(perm_v, i)], wsem[p])

    def drain_write(i):
        p = i % 2
        pltpu.make_async_copy(stage[p], g1_hbm.at[0], wsem[p]).wait()

    for i in range(LPW + 1):
        if i < LPW:
            if i >= 2:
                drain_write(i - 2)
            issue_gather(i)
        if i >= 1:
            wait_gather((i - 1) % 2)
            issue_write(i - 1)
    drain_write(LPW - 2)
    drain_write(LPW - 1)


def _tc_body(g1_ref, ctx_ref, out_ref):
    out_ref[:, 0:1, :] = g1_ref[:, 0:1, :]
    out_ref[:, 1:1 + N_CTX, :] = jnp.broadcast_to(
        ctx_ref[...][None], (TCB, N_CTX, CTX_DIM))
    out_ref[:, 1 + N_CTX:, :] = g1_ref[:, 1:1 + SUF, :]


def kernel(labels, token_embedding, tokenized_prompts, ctx):
    tp1d = jnp.concatenate(
        [tokenized_prompts[:, :1],
         tokenized_prompts[:, 1 + N_CTX:],
         jnp.zeros((tokenized_prompts.shape[0], 3), jnp.int32)],
        axis=1).reshape(-1)
    order = jnp.argsort(labels).astype(jnp.int32)
    labels_sorted = jnp.take(labels, order)
    mesh = plsc.VectorSubcoreMesh(core_axis_name="c", subcore_axis_name="s")
    sc = functools.partial(
        pl.kernel,
        out_type=jax.ShapeDtypeStruct((B, GW, CTX_DIM), jnp.float32),
        mesh=mesh,
        scratch_types=[
            pltpu.VMEM((LPW,), jnp.int32),
            pltpu.VMEM((LPW,), jnp.int32),
            pltpu.VMEM((GW,), jnp.int32),
            pltpu.VMEM((GW,), jnp.int32),
            pltpu.VMEM((GW, CTX_DIM), jnp.float32),
            pltpu.VMEM((GW, CTX_DIM), jnp.float32),
            pltpu.SemaphoreType.DMA,
            pltpu.SemaphoreType.DMA,
            pltpu.SemaphoreType.DMA,
            pltpu.SemaphoreType.DMA,
        ],
        compiler_params=pltpu.CompilerParams(needs_layout_passes=False),
    )(_sc_body)
    g1 = sc(labels_sorted, order, token_embedding, tp1d)
    splice = pl.pallas_call(
        _tc_body,
        grid=(B // TCB,),
        in_specs=[
            pl.BlockSpec((TCB, GW, CTX_DIM), lambda i: (i, 0, 0)),
            pl.BlockSpec((N_CTX, CTX_DIM), lambda i: (0, 0)),
        ],
        out_specs=pl.BlockSpec((TCB, CONTEXT_LEN, CTX_DIM),
                               lambda i: (i, 0, 0)),
        out_shape=jax.ShapeDtypeStruct((B, CONTEXT_LEN, CTX_DIM),
                                       jnp.float32),
    )
    return splice(g1, ctx)


# speedup vs baseline: 1.0811x; 1.0811x over previous
import functools

import jax
import jax.numpy as jnp
from jax import lax
from jax.experimental import pallas as pl
from jax.experimental.pallas import tpu as pltpu
from jax.experimental.pallas import tpu_sc as plsc

B = 1024
CONTEXT_LEN = 77
CTX_DIM = 512
N_CTX = 8
NC, NS = 2, 16
NW = NC * NS
LPW = B // NW
SUF = CONTEXT_LEN - N_CTX - 1
GW = SUF + 4
TCB = 16


def _sc_body(labels_hbm, table_hbm, tp1d_hbm, g1_hbm,
             labels_v, tok0, tok1, s0, s1, gsem0, gsem1, wsem0, wsem1):
    toks = (tok0, tok1)
    stage = (s0, s1)
    gsem = (gsem0, gsem1)
    wsem = (wsem0, wsem1)
    wid = lax.axis_index("s") * NC + lax.axis_index("c")
    base = wid * LPW
    pltpu.sync_copy(labels_hbm.at[pl.ds(base, LPW)], labels_v)
    lanes = lax.iota(jnp.int32, 16)

    def issue_gather(i):
        p = i % 2
        chunk = labels_v[pl.ds(16 * (i // 16), 16)]
        lbl = jnp.max(jnp.where(lanes == (i % 16), chunk, 0))
        pltpu.sync_copy(tp1d_hbm.at[pl.ds(lbl * GW, GW)], toks[p])
        pltpu.async_copy(table_hbm.at[toks[p]], stage[p], gsem[p])

    def wait_gather(p):
        pltpu.make_async_copy(table_hbm.at[pl.ds(0, GW)], stage[p],
                              gsem[p]).wait()

    def issue_write(i):
        p = i % 2
        pltpu.async_copy(stage[p], g1_hbm.at[base + i], wsem[p])

    def drain_write(i):
        p = i % 2
        pltpu.make_async_copy(stage[p], g1_hbm.at[base + i], wsem[p]).wait()

    for i in range(LPW + 1):
        if i < LPW:
            if i >= 2:
                drain_write(i - 2)
            issue_gather(i)
        if i >= 1:
            wait_gather((i - 1) % 2)
            issue_write(i - 1)
    drain_write(LPW - 2)
    drain_write(LPW - 1)


def _tc_body(g1_ref, ctx_ref, out_ref):
    out_ref[:, 0:1, :] = g1_ref[:, 0:1, :]
    out_ref[:, 1:1 + N_CTX, :] = jnp.broadcast_to(
        ctx_ref[...][None], (TCB, N_CTX, CTX_DIM))
    out_ref[:, 1 + N_CTX:, :] = g1_ref[:, 1:1 + SUF, :]


def kernel(labels, token_embedding, tokenized_prompts, ctx):
    tp1d = jnp.concatenate(
        [tokenized_prompts[:, :1],
         tokenized_prompts[:, 1 + N_CTX:],
         jnp.zeros((tokenized_prompts.shape[0], 3), jnp.int32)],
        axis=1).reshape(-1)
    mesh = plsc.VectorSubcoreMesh(core_axis_name="c", subcore_axis_name="s")
    sc = functools.partial(
        pl.kernel,
        out_type=jax.ShapeDtypeStruct((B, GW, CTX_DIM), jnp.float32),
        mesh=mesh,
        scratch_types=[
            pltpu.VMEM((LPW,), jnp.int32),
            pltpu.VMEM((GW,), jnp.int32),
            pltpu.VMEM((GW,), jnp.int32),
            pltpu.VMEM((GW, CTX_DIM), jnp.float32),
            pltpu.VMEM((GW, CTX_DIM), jnp.float32),
            pltpu.SemaphoreType.DMA,
            pltpu.SemaphoreType.DMA,
            pltpu.SemaphoreType.DMA,
            pltpu.SemaphoreType.DMA,
        ],
        compiler_params=pltpu.CompilerParams(needs_layout_passes=False),
    )(_sc_body)
    g1 = sc(labels, token_embedding, tp1d)
    splice = pl.pallas_call(
        _tc_body,
        grid=(B // TCB,),
        in_specs=[
            pl.BlockSpec((TCB, GW, CTX_DIM), lambda i: (i, 0, 0)),
            pl.BlockSpec((N_CTX, CTX_DIM), lambda i: (0, 0)),
        ],
        out_specs=pl.BlockSpec((TCB, CONTEXT_LEN, CTX_DIM),
                               lambda i: (i, 0, 0)),
        out_shape=jax.ShapeDtypeStruct((B, CONTEXT_LEN, CTX_DIM),
                                       jnp.float32),
    )
    return splice(g1, ctx)


# speedup vs baseline: 1.0907x; 1.0089x over previous
import functools

import jax
import jax.numpy as jnp
from jax import lax
from jax.experimental import pallas as pl
from jax.experimental.pallas import tpu as pltpu
from jax.experimental.pallas import tpu_sc as plsc

B = 1024
CONTEXT_LEN = 77
CTX_DIM = 512
N_CTX = 8
NC, NS = 2, 16
NW = NC * NS
LPW = B // NW
SUF = CONTEXT_LEN - N_CTX - 1
GW = SUF + 4
TCB = 32


def _sc_body(labels_hbm, table_hbm, tp1d_hbm, g1_hbm,
             labels_v, tok0, tok1, s0, s1, gsem0, gsem1, wsem0, wsem1):
    toks = (tok0, tok1)
    stage = (s0, s1)
    gsem = (gsem0, gsem1)
    wsem = (wsem0, wsem1)
    wid = lax.axis_index("s") * NC + lax.axis_index("c")
    base = wid * LPW
    pltpu.sync_copy(labels_hbm.at[pl.ds(base, LPW)], labels_v)
    lanes = lax.iota(jnp.int32, 16)

    def issue_gather(i):
        p = i % 2
        chunk = labels_v[pl.ds(16 * (i // 16), 16)]
        lbl = jnp.max(jnp.where(lanes == (i % 16), chunk, 0))
        pltpu.sync_copy(tp1d_hbm.at[pl.ds(lbl * GW, GW)], toks[p])
        pltpu.async_copy(table_hbm.at[toks[p]], stage[p], gsem[p])

    def wait_gather(p):
        pltpu.make_async_copy(table_hbm.at[pl.ds(0, GW)], stage[p],
                              gsem[p]).wait()

    def issue_write(i):
        p = i % 2
        pltpu.async_copy(stage[p], g1_hbm.at[base + i], wsem[p])

    def drain_write(i):
        p = i % 2
        pltpu.make_async_copy(stage[p], g1_hbm.at[base + i], wsem[p]).wait()

    for i in range(LPW + 1):
        if i < LPW:
            if i >= 2:
                drain_write(i - 2)
            issue_gather(i)
        if i >= 1:
            wait_gather((i - 1) % 2)
            issue_write(i - 1)
    drain_write(LPW - 2)
    drain_write(LPW - 1)


def _tc_body(g1_ref, ctx_ref, out_ref):
    out_ref[:, 0:1, :] = g1_ref[:, 0:1, :]
    out_ref[:, 1:1 + N_CTX, :] = jnp.broadcast_to(
        ctx_ref[...][None], (TCB, N_CTX, CTX_DIM))
    out_ref[:, 1 + N_CTX:, :] = g1_ref[:, 1:1 + SUF, :]


def kernel(labels, token_embedding, tokenized_prompts, ctx):
    tp1d = jnp.concatenate(
        [tokenized_prompts[:, :1],
         tokenized_prompts[:, 1 + N_CTX:],
         jnp.zeros((tokenized_prompts.shape[0], 3), jnp.int32)],
        axis=1).reshape(-1)
    mesh = plsc.VectorSubcoreMesh(core_axis_name="c", subcore_axis_name="s")
    sc = functools.partial(
        pl.kernel,
        out_type=jax.ShapeDtypeStruct((B, GW, CTX_DIM), jnp.float32),
        mesh=mesh,
        scratch_types=[
            pltpu.VMEM((LPW,), jnp.int32),
            pltpu.VMEM((GW,), jnp.int32),
            pltpu.VMEM((GW,), jnp.int32),
            pltpu.VMEM((GW, CTX_DIM), jnp.float32),
            pltpu.VMEM((GW, CTX_DIM), jnp.float32),
            pltpu.SemaphoreType.DMA,
            pltpu.SemaphoreType.DMA,
            pltpu.SemaphoreType.DMA,
            pltpu.SemaphoreType.DMA,
        ],
        compiler_params=pltpu.CompilerParams(needs_layout_passes=False),
    )(_sc_body)
    g1 = sc(labels, token_embedding, tp1d)
    splice = pl.pallas_call(
        _tc_body,
        grid=(B // TCB,),
        in_specs=[
            pl.BlockSpec((TCB, GW, CTX_DIM), lambda i: (i, 0, 0)),
            pl.BlockSpec((N_CTX, CTX_DIM), lambda i: (0, 0)),
        ],
        out_specs=pl.BlockSpec((TCB, CONTEXT_LEN, CTX_DIM),
                               lambda i: (i, 0, 0)),
        out_shape=jax.ShapeDtypeStruct((B, CONTEXT_LEN, CTX_DIM),
                                       jnp.float32),
    )
    return splice(g1, ctx)


# speedup vs baseline: 1.0928x; 1.0020x over previous
import functools

import jax
import jax.numpy as jnp
from jax import lax
from jax.experimental import pallas as pl
from jax.experimental.pallas import tpu as pltpu
from jax.experimental.pallas import tpu_sc as plsc

B = 1024
CONTEXT_LEN = 77
CTX_DIM = 512
N_CTX = 8
NC, NS = 2, 16
NW = NC * NS
LPW = B // NW
SUF = CONTEXT_LEN - N_CTX - 1
GW = SUF + 4
TCB = 64


def _sc_body(labels_hbm, table_hbm, tp1d_hbm, g1_hbm,
             labels_v, tok0, tok1, s0, s1, gsem0, gsem1, wsem0, wsem1):
    toks = (tok0, tok1)
    stage = (s0, s1)
    gsem = (gsem0, gsem1)
    wsem = (wsem0, wsem1)
    wid = lax.axis_index("s") * NC + lax.axis_index("c")
    base = wid * LPW
    pltpu.sync_copy(labels_hbm.at[pl.ds(base, LPW)], labels_v)
    lanes = lax.iota(jnp.int32, 16)

    def issue_gather(i):
        p = i % 2
        chunk = labels_v[pl.ds(16 * (i // 16), 16)]
        lbl = jnp.max(jnp.where(lanes == (i % 16), chunk, 0))
        pltpu.sync_copy(tp1d_hbm.at[pl.ds(lbl * GW, GW)], toks[p])
        pltpu.async_copy(table_hbm.at[toks[p]], stage[p], gsem[p])

    def wait_gather(p):
        pltpu.make_async_copy(table_hbm.at[pl.ds(0, GW)], stage[p],
                              gsem[p]).wait()

    def issue_write(i):
        p = i % 2
        pltpu.async_copy(stage[p], g1_hbm.at[base + i], wsem[p])

    def drain_write(i):
        p = i % 2
        pltpu.make_async_copy(stage[p], g1_hbm.at[base + i], wsem[p]).wait()

    for i in range(LPW + 1):
        if i < LPW:
            if i >= 2:
                drain_write(i - 2)
            issue_gather(i)
        if i >= 1:
            wait_gather((i - 1) % 2)
            issue_write(i - 1)
    drain_write(LPW - 2)
    drain_write(LPW - 1)


def _tc_body(g1_ref, ctx_ref, out_ref):
    out_ref[:, 0:1, :] = g1_ref[:, 0:1, :]
    out_ref[:, 1:1 + N_CTX, :] = jnp.broadcast_to(
        ctx_ref[...][None], (TCB, N_CTX, CTX_DIM))
    out_ref[:, 1 + N_CTX:, :] = g1_ref[:, 1:1 + SUF, :]


def kernel(labels, token_embedding, tokenized_prompts, ctx):
    tp1d = jnp.concatenate(
        [tokenized_prompts[:, :1],
         tokenized_prompts[:, 1 + N_CTX:],
         jnp.zeros((tokenized_prompts.shape[0], 3), jnp.int32)],
        axis=1).reshape(-1)
    mesh = plsc.VectorSubcoreMesh(core_axis_name="c", subcore_axis_name="s")
    sc = functools.partial(
        pl.kernel,
        out_type=jax.ShapeDtypeStruct((B, GW, CTX_DIM), jnp.float32),
        mesh=mesh,
        scratch_types=[
            pltpu.VMEM((LPW,), jnp.int32),
            pltpu.VMEM((GW,), jnp.int32),
            pltpu.VMEM((GW,), jnp.int32),
            pltpu.VMEM((GW, CTX_DIM), jnp.float32),
            pltpu.VMEM((GW, CTX_DIM), jnp.float32),
            pltpu.SemaphoreType.DMA,
            pltpu.SemaphoreType.DMA,
            pltpu.SemaphoreType.DMA,
            pltpu.SemaphoreType.DMA,
        ],
        compiler_params=pltpu.CompilerParams(needs_layout_passes=False),
    )(_sc_body)
    g1 = sc(labels, token_embedding, tp1d)
    splice = pl.pallas_call(
        _tc_body,
        grid=(B // TCB,),
        in_specs=[
            pl.BlockSpec((TCB, GW, CTX_DIM), lambda i: (i, 0, 0)),
            pl.BlockSpec((N_CTX, CTX_DIM), lambda i: (0, 0)),
        ],
        out_specs=pl.BlockSpec((TCB, CONTEXT_LEN, CTX_DIM),
                               lambda i: (i, 0, 0)),
        out_shape=jax.ShapeDtypeStruct((B, CONTEXT_LEN, CTX_DIM),
                                       jnp.float32),
    )
    return splice(g1, ctx)


# speedup vs baseline: 1.0965x; 1.0033x over previous
import functools

import jax
import jax.numpy as jnp
from jax import lax
from jax.experimental import pallas as pl
from jax.experimental.pallas import tpu as pltpu
from jax.experimental.pallas import tpu_sc as plsc

B = 1024
CONTEXT_LEN = 77
CTX_DIM = 512
N_CTX = 8
NC, NS = 2, 16
NW = NC * NS
LPW = B // NW
SUF = CONTEXT_LEN - N_CTX - 1
GW = SUF + 4
TCB = 64


def _sc_body(labels_hbm, table_hbm, tp1d_hbm, g1_hbm,
             labels_v, tok0, tok1, s0, s1, gsem0, gsem1,
             hsem0, hsem1, wsem0, wsem1):
    toks = (tok0, tok1)
    stage = (s0, s1)
    gsem = (gsem0, gsem1)
    hsem = (hsem0, hsem1)
    wsem = (wsem0, wsem1)
    wid = lax.axis_index("s") * NC + lax.axis_index("c")
    base = wid * LPW
    pltpu.sync_copy(labels_hbm.at[pl.ds(base, LPW)], labels_v)
    lanes = lax.iota(jnp.int32, 16)

    def issue_gather(i):
        p = i % 2
        chunk = labels_v[pl.ds(16 * (i // 16), 16)]
        lbl = jnp.max(jnp.where(lanes == (i % 16), chunk, 0))
        pltpu.sync_copy(tp1d_hbm.at[pl.ds(lbl * GW, GW)], toks[p])
        pltpu.async_copy(table_hbm.at[toks[p].at[pl.ds(0, 40)]],
                         stage[p].at[pl.ds(0, 40)], gsem[p])
        pltpu.async_copy(table_hbm.at[toks[p].at[pl.ds(40, 32)]],
                         stage[p].at[pl.ds(40, 32)], hsem[p])

    def wait_gather(p):
        pltpu.make_async_copy(table_hbm.at[pl.ds(0, 40)],
                              stage[p].at[pl.ds(0, 40)], gsem[p]).wait()
        pltpu.make_async_copy(table_hbm.at[pl.ds(0, 32)],
                              stage[p].at[pl.ds(40, 32)], hsem[p]).wait()

    def issue_write(i):
        p = i % 2
        pltpu.async_copy(stage[p], g1_hbm.at[base + i], wsem[p])

    def drain_write(i):
        p = i % 2
        pltpu.make_async_copy(stage[p], g1_hbm.at[base + i], wsem[p]).wait()

    for i in range(LPW + 1):
        if i < LPW:
            if i >= 2:
                drain_write(i - 2)
            issue_gather(i)
        if i >= 1:
            wait_gather((i - 1) % 2)
            issue_write(i - 1)
    drain_write(LPW - 2)
    drain_write(LPW - 1)


def _tc_body(g1_ref, ctx_ref, out_ref):
    out_ref[:, 0:1, :] = g1_ref[:, 0:1, :]
    out_ref[:, 1:1 + N_CTX, :] = jnp.broadcast_to(
        ctx_ref[...][None], (TCB, N_CTX, CTX_DIM))
    out_ref[:, 1 + N_CTX:, :] = g1_ref[:, 1:1 + SUF, :]


def kernel(labels, token_embedding, tokenized_prompts, ctx):
    tp1d = jnp.concatenate(
        [tokenized_prompts[:, :1],
         tokenized_prompts[:, 1 + N_CTX:],
         jnp.zeros((tokenized_prompts.shape[0], 3), jnp.int32)],
        axis=1).reshape(-1)
    mesh = plsc.VectorSubcoreMesh(core_axis_name="c", subcore_axis_name="s")
    sc = functools.partial(
        pl.kernel,
        out_type=jax.ShapeDtypeStruct((B, GW, CTX_DIM), jnp.float32),
        mesh=mesh,
        scratch_types=[
            pltpu.VMEM((LPW,), jnp.int32),
            pltpu.VMEM((GW,), jnp.int32),
            pltpu.VMEM((GW,), jnp.int32),
            pltpu.VMEM((GW, CTX_DIM), jnp.float32),
            pltpu.VMEM((GW, CTX_DIM), jnp.float32),
            pltpu.SemaphoreType.DMA,
            pltpu.SemaphoreType.DMA,
            pltpu.SemaphoreType.DMA,
            pltpu.SemaphoreType.DMA,
            pltpu.SemaphoreType.DMA,
            pltpu.SemaphoreType.DMA,
        ],
        compiler_params=pltpu.CompilerParams(needs_layout_passes=False),
    )(_sc_body)
    g1 = sc(labels, token_embedding, tp1d)
    splice = pl.pallas_call(
        _tc_body,
        grid=(B // TCB,),
        in_specs=[
            pl.BlockSpec((TCB, GW, CTX_DIM), lambda i: (i, 0, 0)),
            pl.BlockSpec((N_CTX, CTX_DIM), lambda i: (0, 0)),
        ],
        out_specs=pl.BlockSpec((TCB, CONTEXT_LEN, CTX_DIM),
                               lambda i: (i, 0, 0)),
        out_shape=jax.ShapeDtypeStruct((B, CONTEXT_LEN, CTX_DIM),
                                       jnp.float32),
    )
    return splice(g1, ctx)
